# baseline (speedup 1.0000x reference)
import functools

import jax
import jax.numpy as jnp
from jax import lax
from jax.experimental import pallas as pl
from jax.experimental.pallas import tpu as pltpu
from jax.experimental.pallas import tpu_sc as plsc

NN = 50000
EE = 800000
DD = 64
HF = DD // 2
GG = 128
NC = 2
NS = 16
LANES = 16

NP = 51200
ECH = 80
SB = 5
CPT = EE // ECH // NS
SBT = CPT // SB
RPT = NN // NS
IWCH = RPT // 5

_MESH = plsc.VectorSubcoreMesh(
    core_axis_name="c", subcore_axis_name="s", num_cores=NC, num_subcores=NS)
_SC_PARAMS = pltpu.CompilerParams(use_tc_tiling_on_sc=False)


def _agg_body(h_hbm, src_hbm, dst_hbm, out_hbm, acc, idxs, idxd, rows,
              gsem, ssem):
  c = lax.axis_index("c")
  s = lax.axis_index("s")
  cbase = c * NP

  for k in range(RPT // IWCH):
    pltpu.sync_copy(
        h_hbm.at[pl.ds(cbase + s * RPT + k * IWCH, IWCH)],
        acc.at[pl.ds(s * RPT + k * IWCH, IWCH)])
  plsc.subcore_barrier()

  cb = s * CPT

  def load_idx(t, slot):
    pltpu.sync_copy(src_hbm.at[pl.ds(cb + t * SB, SB)], idxs.at[slot])
    pltpu.sync_copy(dst_hbm.at[pl.ds(cb + t * SB, SB)], idxd.at[slot])
    for j in range(SB):
      for k in range(ECH // LANES):
        v = idxs[slot, j, pl.ds(k * LANES, LANES)]
        idxs[slot, j, pl.ds(k * LANES, LANES)] = v + cbase

  def fire_gathers(slot):
    for j in range(SB):
      pltpu.async_copy(h_hbm.at[idxs.at[slot, j]], rows.at[slot, j], gsem)

  def drain_gather(slot, j):
    pltpu.make_async_copy(h_hbm.at[pl.ds(0, ECH)], rows.at[slot, j],
                          gsem).wait()

  def drain_scatter(slot, j):
    pltpu.make_async_copy(rows.at[slot, j], acc.at[pl.ds(0, ECH)],
                          ssem).wait()

  load_idx(0, 0)
  fire_gathers(0)

  def sb_body(t, carry):
    p = lax.rem(t, 2)
    q = 1 - p

    @pl.when(t >= 1)
    def _():
      for j in range(SB):
        drain_scatter(q, j)

    @pl.when(t < SBT - 1)
    def _():
      load_idx(t + 1, q)
      fire_gathers(q)

    for j in range(SB):
      drain_gather(p, j)
      pltpu.async_copy(rows.at[p, j], acc.at[idxd.at[p, j]], ssem, add=True)
    return carry

  lax.fori_loop(0, SBT, sb_body, 0)
  for j in range(SB):
    drain_scatter((SBT - 1) % 2, j)
  plsc.subcore_barrier()

  for k in range(RPT // IWCH):
    pltpu.sync_copy(
        acc.at[pl.ds(s * RPT + k * IWCH, IWCH)],
        out_hbm.at[pl.ds(cbase + s * RPT + k * IWCH, IWCH)])


_agg = pl.kernel(
    _agg_body,
    out_type=jax.ShapeDtypeStruct((NC * NP, HF), jnp.float32),
    mesh=_MESH,
    compiler_params=_SC_PARAMS,
    scratch_types=[
        pltpu.VMEM_SHARED((NN, HF), jnp.float32),
        pltpu.VMEM((2, SB, ECH), jnp.int32),
        pltpu.VMEM((2, SB, ECH), jnp.int32),
        pltpu.VMEM((2, SB, ECH, HF), jnp.float32),
        pltpu.SemaphoreType.DMA,
        pltpu.SemaphoreType.DMA,
    ],
)

PCH = 80
NPCH = NN // PCH
GPT = GG // NS


def _pool_body(h_hbm, batch_hbm, out_hbm, acc, idx_b, rows):
  c = lax.axis_index("c")
  s = lax.axis_index("s")

  zero = jnp.zeros((LANES,), jnp.float32)
  for i in range(GPT):
    for k in range(HF // LANES):
      rows[i, pl.ds(k * LANES, LANES)] = zero
  pltpu.sync_copy(rows.at[pl.ds(0, GPT)], acc.at[pl.ds(s * GPT, GPT)])
  plsc.subcore_barrier()

  def rchunk(i, carry):
    q = s * 40 + i

    @pl.when(q < NPCH)
    def _():
      pltpu.sync_copy(h_hbm.at[pl.ds(c * NP + q * PCH, PCH)], rows)
      pltpu.sync_copy(batch_hbm.at[pl.ds(q * PCH, PCH)], idx_b)
      pltpu.sync_copy(rows, acc.at[idx_b], add=True)

    return carry

  lax.fori_loop(0, 40, rchunk, 0)
  plsc.subcore_barrier()

  @pl.when(s == 0)
  def _():
    pltpu.sync_copy(acc, out_hbm.at[c])


_pool = pl.kernel(
    _pool_body,
    out_type=jax.ShapeDtypeStruct((NC, GG, HF), jnp.float32),
    mesh=_MESH,
    compiler_params=_SC_PARAMS,
    scratch_types=[
        pltpu.VMEM_SHARED((GG, HF), jnp.float32),
        pltpu.VMEM((PCH,), jnp.int32),
        pltpu.VMEM((PCH, HF), jnp.float32),
    ],
)

NQ = NC * NP * HF // 128
QB = 512
NRB = NQ // NC // QB


def _mlp_body(zl_ref, zh_ref, w1ll_ref, w1hl_ref, w1lh_ref, w1hh_ref,
              b1l_ref, b1h_ref, w2ll_ref, w2hl_ref, w2lh_ref, w2hh_ref,
              b2l_ref, b2h_ref, o_ref):
  j = pl.program_id(1)
  zl = zl_ref[...]
  zh = zh_ref[...]
  tl = jnp.dot(zl, w1ll_ref[...], preferred_element_type=jnp.float32) \
      + jnp.dot(zh, w1hl_ref[...], preferred_element_type=jnp.float32)
  tl = jnp.maximum(tl + b1l_ref[...], 0.0)
  th = jnp.dot(zl, w1lh_ref[...], preferred_element_type=jnp.float32) \
      + jnp.dot(zh, w1hh_ref[...], preferred_element_type=jnp.float32)
  th = jnp.maximum(th + b1h_ref[...], 0.0)
  w2l = jnp.where(j == 0, w2ll_ref[...], w2lh_ref[...])
  w2h = jnp.where(j == 0, w2hl_ref[...], w2hh_ref[...])
  b2 = jnp.where(j == 0, b2l_ref[...], b2h_ref[...])
  o = jnp.dot(tl, w2l, preferred_element_type=jnp.float32) \
      + jnp.dot(th, w2h, preferred_element_type=jnp.float32)
  o_ref[...] = jnp.maximum(o + b2, 0.0)


def _bd4(m):
  return jnp.kron(jnp.eye(4, dtype=m.dtype), m)


def _tile4(v):
  return jnp.tile(v, 4).reshape(1, 128)


def _mlp(z2, W1, b1, W2, b2):
  zv = z2.reshape(NQ, 128)
  full = lambda i, j: (0, 0)
  out = pl.pallas_call(
      _mlp_body,
      grid=(NRB, NC),
      in_specs=[
          pl.BlockSpec((QB, 128), lambda i, j: (i, 0)),
          pl.BlockSpec((QB, 128), lambda i, j: (NRB + i, 0)),
          pl.BlockSpec((128, 128), full),
          pl.BlockSpec((128, 128), full),
          pl.BlockSpec((128, 128), full),
          pl.BlockSpec((128, 128), full),
          pl.BlockSpec((1, 128), full),
          pl.BlockSpec((1, 128), full),
          pl.BlockSpec((128, 128), full),
          pl.BlockSpec((128, 128), full),
          pl.BlockSpec((128, 128), full),
          pl.BlockSpec((128, 128), full),
          pl.BlockSpec((1, 128), full),
          pl.BlockSpec((1, 128), full),
      ],
      out_specs=pl.BlockSpec((QB, 128), lambda i, j: (j * NRB + i, 0)),
      out_shape=jax.ShapeDtypeStruct((NQ, 128), jnp.float32),
  )(zv, zv,
    _bd4(W1[:HF, :HF]), _bd4(W1[HF:, :HF]),
    _bd4(W1[:HF, HF:]), _bd4(W1[HF:, HF:]),
    _tile4(b1[:HF]), _tile4(b1[HF:]),
    _bd4(W2[:HF, :HF]), _bd4(W2[HF:, :HF]),
    _bd4(W2[:HF, HF:]), _bd4(W2[HF:, HF:]),
    _tile4(b2[:HF]), _tile4(b2[HF:]))
  return out.reshape(NC * NP, HF)


def _final_body(p_ref, w1_ref, b1_ref, w2_ref, b2_ref, o_ref):
  g = jnp.concatenate([p_ref[0], p_ref[1]], axis=1)
  t = jnp.dot(g, w1_ref[...], preferred_element_type=jnp.float32)
  t = jnp.maximum(t + b1_ref[...], 0.0)
  o_ref[...] = jnp.dot(t, w2_ref[...], preferred_element_type=jnp.float32) \
      + b2_ref[...]


def _final(parts, W1, b1, W2, b2):
  return pl.pallas_call(
      _final_body,
      out_shape=jax.ShapeDtypeStruct((GG, 1), jnp.float32),
  )(parts, W1, b1.reshape(1, DD), W2, b2.reshape(1, 1))


@jax.jit
def kernel(x, edge_index, batch, params):
  src2 = edge_index[0].reshape(EE // ECH, ECH)
  dst2 = edge_index[1].reshape(EE // ECH, ECH)
  zpad = jnp.zeros((NP - NN, HF), jnp.float32)
  h2 = jnp.concatenate([x[:, :HF], zpad, x[:, HF:], zpad], axis=0)
  for (W1, b1, W2, b2) in params["convs"]:
    z2 = _agg(h2, src2, dst2)
    h2 = _mlp(z2, W1, b1, W2, b2)
  parts = _pool(h2, batch)
  out = _final(parts, params["fc_W1"], params["fc_b1"],
               params["fc_W2"], params["fc_b2"])
  return out[:, 0]


# speedup vs baseline: 1.0125x; 1.0125x over previous
import functools

import jax
import jax.numpy as jnp
from jax import lax
from jax.experimental import pallas as pl
from jax.experimental.pallas import tpu as pltpu
from jax.experimental.pallas import tpu_sc as plsc

NN = 50000
EE = 800000
DD = 64
HF = DD // 2
GG = 128
NC = 2
NS = 16
LANES = 16

NP = 51200
ECH = 80
SB = 5
CPT = EE // ECH // NS
SBT = CPT // SB
RPT = NN // NS
IWCH = RPT // 5

_MESH = plsc.VectorSubcoreMesh(
    core_axis_name="c", subcore_axis_name="s", num_cores=NC, num_subcores=NS)
_SC_PARAMS = pltpu.CompilerParams(use_tc_tiling_on_sc=False)


def _agg_body(h_hbm, src_hbm, dst_hbm, out_hbm, acc, idxs, idxd, rows,
              gsem, ssem):
  c = lax.axis_index("c")
  s = lax.axis_index("s")
  cbase = c * NP

  for k in range(RPT // IWCH):
    pltpu.sync_copy(
        h_hbm.at[pl.ds(cbase + s * RPT + k * IWCH, IWCH)],
        acc.at[pl.ds(s * RPT + k * IWCH, IWCH)])
  plsc.subcore_barrier()

  cb = s * CPT

  def load_idx(t, slot):
    pltpu.sync_copy(src_hbm.at[pl.ds(cb + t * SB, SB)], idxs.at[slot])
    pltpu.sync_copy(dst_hbm.at[pl.ds(cb + t * SB, SB)], idxd.at[slot])
    for j in range(SB):
      for k in range(ECH // LANES):
        v = idxs[slot, j, pl.ds(k * LANES, LANES)]
        idxs[slot, j, pl.ds(k * LANES, LANES)] = v + cbase

  def fire_gathers(slot):
    for j in range(SB):
      pltpu.async_copy(h_hbm.at[idxs.at[slot, j]], rows.at[slot, j], gsem)

  def drain_gather(slot, j):
    pltpu.make_async_copy(h_hbm.at[pl.ds(0, ECH)], rows.at[slot, j],
                          gsem).wait()

  def drain_scatter(slot, j):
    pltpu.make_async_copy(rows.at[slot, j], acc.at[pl.ds(0, ECH)],
                          ssem).wait()

  load_idx(0, 0)
  fire_gathers(0)

  def sb_body(t, carry):
    p = lax.rem(t, 2)
    q = 1 - p

    @pl.when(t >= 1)
    def _():
      for j in range(SB):
        drain_scatter(q, j)

    @pl.when(t < SBT - 1)
    def _():
      load_idx(t + 1, q)
      fire_gathers(q)

    for j in range(SB):
      drain_gather(p, j)
      pltpu.async_copy(rows.at[p, j], acc.at[idxd.at[p, j]], ssem, add=True)
    return carry

  lax.fori_loop(0, SBT, sb_body, 0)
  for j in range(SB):
    drain_scatter((SBT - 1) % 2, j)
  plsc.subcore_barrier()

  for k in range(RPT // IWCH):
    pltpu.sync_copy(
        acc.at[pl.ds(s * RPT + k * IWCH, IWCH)],
        out_hbm.at[pl.ds(cbase + s * RPT + k * IWCH, IWCH)])


_agg = pl.kernel(
    _agg_body,
    out_type=jax.ShapeDtypeStruct((NC * NP, HF), jnp.float32),
    mesh=_MESH,
    compiler_params=_SC_PARAMS,
    scratch_types=[
        pltpu.VMEM_SHARED((NN, HF), jnp.float32),
        pltpu.VMEM((2, SB, ECH), jnp.int32),
        pltpu.VMEM((2, SB, ECH), jnp.int32),
        pltpu.VMEM((2, SB, ECH, HF), jnp.float32),
        pltpu.SemaphoreType.DMA,
        pltpu.SemaphoreType.DMA,
    ],
)

PCH = 80
NPCH = NN // PCH
GPT = GG // NS


def _pool_body(h_hbm, batch_hbm, out_hbm, acc, idx_b, rows):
  c = lax.axis_index("c")
  s = lax.axis_index("s")

  zero = jnp.zeros((LANES,), jnp.float32)
  for i in range(GPT):
    for k in range(HF // LANES):
      rows[i, pl.ds(k * LANES, LANES)] = zero
  pltpu.sync_copy(rows.at[pl.ds(0, GPT)], acc.at[pl.ds(s * GPT, GPT)])
  plsc.subcore_barrier()

  def rchunk(i, carry):
    q = s * 40 + i

    @pl.when(q < NPCH)
    def _():
      pltpu.sync_copy(h_hbm.at[pl.ds(c * NP + q * PCH, PCH)], rows)
      pltpu.sync_copy(batch_hbm.at[pl.ds(q * PCH, PCH)], idx_b)
      pltpu.sync_copy(rows, acc.at[idx_b], add=True)

    return carry

  lax.fori_loop(0, 40, rchunk, 0)
  plsc.subcore_barrier()

  @pl.when(s == 0)
  def _():
    pltpu.sync_copy(acc, out_hbm.at[c])


_pool = pl.kernel(
    _pool_body,
    out_type=jax.ShapeDtypeStruct((NC, GG, HF), jnp.float32),
    mesh=_MESH,
    compiler_params=_SC_PARAMS,
    scratch_types=[
        pltpu.VMEM_SHARED((GG, HF), jnp.float32),
        pltpu.VMEM((PCH,), jnp.int32),
        pltpu.VMEM((PCH, HF), jnp.float32),
    ],
)

NQ = NC * NP * HF // 128
QB = 512
NRB = NQ // NC // QB


def _mlp_body(zl_ref, zh_ref, w1_ref, b1_ref, w2l_ref, w2h_ref,
              b2l_ref, b2h_ref, o_ref):
  j = pl.program_id(1)
  z = jnp.concatenate([zl_ref[...], zh_ref[...]], axis=1)
  t = jnp.dot(z, w1_ref[...], preferred_element_type=jnp.float32)
  t = jnp.maximum(t + b1_ref[...], 0.0)
  w2 = jnp.where(j == 0, w2l_ref[...], w2h_ref[...])
  b2 = jnp.where(j == 0, b2l_ref[...], b2h_ref[...])
  o = jnp.dot(t, w2, preferred_element_type=jnp.float32)
  o_ref[...] = jnp.maximum(o + b2, 0.0)


def _bd4(m):
  return jnp.kron(jnp.eye(4, dtype=m.dtype), m)


def _tile4(v):
  return jnp.tile(v, 4).reshape(1, 128)


def _mlp(z2, W1, b1, W2, b2):
  zv = z2.reshape(NQ, 128)
  full = lambda i, j: (0, 0)
  w1big = jnp.concatenate([
      jnp.concatenate([_bd4(W1[:HF, :HF]), _bd4(W1[:HF, HF:])], axis=1),
      jnp.concatenate([_bd4(W1[HF:, :HF]), _bd4(W1[HF:, HF:])], axis=1),
  ], axis=0)
  b1big = jnp.concatenate([_tile4(b1[:HF]), _tile4(b1[HF:])], axis=1)
  w2lo = jnp.concatenate([_bd4(W2[:HF, :HF]), _bd4(W2[HF:, :HF])], axis=0)
  w2hi = jnp.concatenate([_bd4(W2[:HF, HF:]), _bd4(W2[HF:, HF:])], axis=0)
  out = pl.pallas_call(
      _mlp_body,
      grid=(NRB, NC),
      in_specs=[
          pl.BlockSpec((QB, 128), lambda i, j: (i, 0)),
          pl.BlockSpec((QB, 128), lambda i, j: (NRB + i, 0)),
          pl.BlockSpec((256, 256), full),
          pl.BlockSpec((1, 256), full),
          pl.BlockSpec((256, 128), full),
          pl.BlockSpec((256, 128), full),
          pl.BlockSpec((1, 128), full),
          pl.BlockSpec((1, 128), full),
      ],
      out_specs=pl.BlockSpec((QB, 128), lambda i, j: (j * NRB + i, 0)),
      out_shape=jax.ShapeDtypeStruct((NQ, 128), jnp.float32),
  )(zv, zv, w1big, b1big, w2lo, w2hi, _tile4(b2[:HF]), _tile4(b2[HF:]))
  return out.reshape(NC * NP, HF)


def _final_body(p_ref, w1_ref, b1_ref, w2_ref, b2_ref, o_ref):
  g = jnp.concatenate([p_ref[0], p_ref[1]], axis=1)
  t = jnp.dot(g, w1_ref[...], preferred_element_type=jnp.float32)
  t = jnp.maximum(t + b1_ref[...], 0.0)
  o_ref[...] = jnp.dot(t, w2_ref[...], preferred_element_type=jnp.float32) \
      + b2_ref[...]


def _final(parts, W1, b1, W2, b2):
  return pl.pallas_call(
      _final_body,
      out_shape=jax.ShapeDtypeStruct((GG, 1), jnp.float32),
  )(parts, W1, b1.reshape(1, DD), W2, b2.reshape(1, 1))


@jax.jit
def kernel(x, edge_index, batch, params):
  src2 = edge_index[0].reshape(EE // ECH, ECH)
  dst2 = edge_index[1].reshape(EE // ECH, ECH)
  zpad = jnp.zeros((NP - NN, HF), jnp.float32)
  h2 = jnp.concatenate([x[:, :HF], zpad, x[:, HF:], zpad], axis=0)
  for (W1, b1, W2, b2) in params["convs"]:
    z2 = _agg(h2, src2, dst2)
    h2 = _mlp(z2, W1, b1, W2, b2)
  parts = _pool(h2, batch)
  out = _final(parts, params["fc_W1"], params["fc_b1"],
               params["fc_W2"], params["fc_b2"])
  return out[:, 0]


# speedup vs baseline: 1.4389x; 1.4212x over previous
import functools

import jax
import jax.numpy as jnp
from jax import lax
from jax.experimental import pallas as pl
from jax.experimental.pallas import tpu as pltpu
from jax.experimental.pallas import tpu_sc as plsc

NN = 50000
EE = 800000
DD = 64
HF = DD // 2
GG = 128
NC = 2
NS = 16
LANES = 16

NP = 51200
ECH = 80
SB = 5
CPT = EE // ECH // NS
SBT = CPT // SB
RPT = NN // NS
IWCH = RPT // 5

_MESH = plsc.VectorSubcoreMesh(
    core_axis_name="c", subcore_axis_name="s", num_cores=NC, num_subcores=NS)
_SC_PARAMS = pltpu.CompilerParams(use_tc_tiling_on_sc=False)


def _agg_body(h_hbm, src_hbm, dst_hbm, out_hbm, acc, idxs, idxd, rows,
              gsem, ssem, isem):
  c = lax.axis_index("c")
  s = lax.axis_index("s")
  cbase = c * NP

  for k in range(RPT // IWCH):
    pltpu.sync_copy(
        h_hbm.at[pl.ds(cbase + s * RPT + k * IWCH, IWCH)],
        acc.at[pl.ds(s * RPT + k * IWCH, IWCH)])
  plsc.subcore_barrier()

  cb = s * CPT

  def load_idx_async(t, slot):
    pltpu.async_copy(src_hbm.at[pl.ds(cb + t * SB, SB)], idxs.at[slot], isem)
    pltpu.async_copy(dst_hbm.at[pl.ds(cb + t * SB, SB)], idxd.at[slot], isem)

  def wait_idx(slot):
    pltpu.make_async_copy(src_hbm.at[pl.ds(0, SB)], idxs.at[slot],
                          isem).wait()
    pltpu.make_async_copy(src_hbm.at[pl.ds(0, SB)], idxd.at[slot],
                          isem).wait()

  def shift_idx(slot):
    for j in range(SB):
      for k in range(ECH // LANES):
        v = idxs[slot, j, pl.ds(k * LANES, LANES)]
        idxs[slot, j, pl.ds(k * LANES, LANES)] = v + cbase

  def fire_gathers(islot, rslot):
    for j in range(SB):
      pltpu.async_copy(h_hbm.at[idxs.at[islot, j]], rows.at[rslot, j], gsem)

  def drain_gather(rslot, j):
    pltpu.make_async_copy(h_hbm.at[pl.ds(0, ECH)], rows.at[rslot, j],
                          gsem).wait()

  def drain_scatter(rslot, j):
    pltpu.make_async_copy(rows.at[rslot, j], acc.at[pl.ds(0, ECH)],
                          ssem).wait()

  load_idx_async(0, 0)
  wait_idx(0)
  shift_idx(0)
  fire_gathers(0, 0)
  load_idx_async(1, 1)

  def sb_body(t, carry):
    p = lax.rem(t, 2)
    i0 = lax.rem(t, 3)
    i1 = lax.rem(t + 1, 3)
    i2 = lax.rem(t + 2, 3)

    @pl.when(t >= 1)
    def _():
      for j in range(SB):
        drain_scatter(1 - p, j)

    @pl.when(t < SBT - 1)
    def _():
      wait_idx(i1)
      shift_idx(i1)
      fire_gathers(i1, 1 - p)

    @pl.when(t < SBT - 2)
    def _():
      load_idx_async(t + 2, i2)

    for j in range(SB):
      drain_gather(p, j)
      pltpu.async_copy(rows.at[p, j], acc.at[idxd.at[i0, j]], ssem, add=True)
    return carry

  lax.fori_loop(0, SBT, sb_body, 0)
  for j in range(SB):
    drain_scatter((SBT - 1) % 2, j)
  plsc.subcore_barrier()

  for k in range(RPT // IWCH):
    pltpu.sync_copy(
        acc.at[pl.ds(s * RPT + k * IWCH, IWCH)],
        out_hbm.at[pl.ds(cbase + s * RPT + k * IWCH, IWCH)])


_agg = pl.kernel(
    _agg_body,
    out_type=jax.ShapeDtypeStruct((NC * NP, HF), jnp.float32),
    mesh=_MESH,
    compiler_params=_SC_PARAMS,
    scratch_types=[
        pltpu.VMEM_SHARED((NN, HF), jnp.float32),
        pltpu.VMEM((3, SB, ECH), jnp.int32),
        pltpu.VMEM((3, SB, ECH), jnp.int32),
        pltpu.VMEM((2, SB, ECH, HF), jnp.float32),
        pltpu.SemaphoreType.DMA,
        pltpu.SemaphoreType.DMA,
        pltpu.SemaphoreType.DMA,
    ],
)

PCH = 80
NPCH = NN // PCH
GPT = GG // NS


def _pool_body(h_hbm, batch_hbm, out_hbm, acc, idx_b, rows):
  c = lax.axis_index("c")
  s = lax.axis_index("s")

  zero = jnp.zeros((LANES,), jnp.float32)
  for i in range(GPT):
    for k in range(HF // LANES):
      rows[i, pl.ds(k * LANES, LANES)] = zero
  pltpu.sync_copy(rows.at[pl.ds(0, GPT)], acc.at[pl.ds(s * GPT, GPT)])
  plsc.subcore_barrier()

  def rchunk(i, carry):
    q = s * 40 + i

    @pl.when(q < NPCH)
    def _():
      pltpu.sync_copy(h_hbm.at[pl.ds(c * NP + q * PCH, PCH)], rows)
      pltpu.sync_copy(batch_hbm.at[pl.ds(q * PCH, PCH)], idx_b)
      pltpu.sync_copy(rows, acc.at[idx_b], add=True)

    return carry

  lax.fori_loop(0, 40, rchunk, 0)
  plsc.subcore_barrier()

  @pl.when(s == 0)
  def _():
    pltpu.sync_copy(acc, out_hbm.at[c])


_pool = pl.kernel(
    _pool_body,
    out_type=jax.ShapeDtypeStruct((NC, GG, HF), jnp.float32),
    mesh=_MESH,
    compiler_params=_SC_PARAMS,
    scratch_types=[
        pltpu.VMEM_SHARED((GG, HF), jnp.float32),
        pltpu.VMEM((PCH,), jnp.int32),
        pltpu.VMEM((PCH, HF), jnp.float32),
    ],
)

NQ = NC * NP * HF // 128
QB = 512
NRB = NQ // NC // QB


def _mlp_body(zl_ref, zh_ref, w1_ref, b1_ref, w2l_ref, w2h_ref,
              b2l_ref, b2h_ref, o_ref):
  j = pl.program_id(1)
  z = jnp.concatenate([zl_ref[...], zh_ref[...]], axis=1)
  t = jnp.dot(z, w1_ref[...], preferred_element_type=jnp.float32)
  t = jnp.maximum(t + b1_ref[...], 0.0)
  w2 = jnp.where(j == 0, w2l_ref[...], w2h_ref[...])
  b2 = jnp.where(j == 0, b2l_ref[...], b2h_ref[...])
  o = jnp.dot(t, w2, preferred_element_type=jnp.float32)
  o_ref[...] = jnp.maximum(o + b2, 0.0)


def _bd4(m):
  return jnp.kron(jnp.eye(4, dtype=m.dtype), m)


def _tile4(v):
  return jnp.tile(v, 4).reshape(1, 128)


def _mlp(z2, W1, b1, W2, b2):
  zv = z2.reshape(NQ, 128)
  full = lambda i, j: (0, 0)
  w1big = jnp.concatenate([
      jnp.concatenate([_bd4(W1[:HF, :HF]), _bd4(W1[:HF, HF:])], axis=1),
      jnp.concatenate([_bd4(W1[HF:, :HF]), _bd4(W1[HF:, HF:])], axis=1),
  ], axis=0)
  b1big = jnp.concatenate([_tile4(b1[:HF]), _tile4(b1[HF:])], axis=1)
  w2lo = jnp.concatenate([_bd4(W2[:HF, :HF]), _bd4(W2[HF:, :HF])], axis=0)
  w2hi = jnp.concatenate([_bd4(W2[:HF, HF:]), _bd4(W2[HF:, HF:])], axis=0)
  out = pl.pallas_call(
      _mlp_body,
      grid=(NRB, NC),
      in_specs=[
          pl.BlockSpec((QB, 128), lambda i, j: (i, 0)),
          pl.BlockSpec((QB, 128), lambda i, j: (NRB + i, 0)),
          pl.BlockSpec((256, 256), full),
          pl.BlockSpec((1, 256), full),
          pl.BlockSpec((256, 128), full),
          pl.BlockSpec((256, 128), full),
          pl.BlockSpec((1, 128), full),
          pl.BlockSpec((1, 128), full),
      ],
      out_specs=pl.BlockSpec((QB, 128), lambda i, j: (j * NRB + i, 0)),
      out_shape=jax.ShapeDtypeStruct((NQ, 128), jnp.float32),
  )(zv, zv, w1big, b1big, w2lo, w2hi, _tile4(b2[:HF]), _tile4(b2[HF:]))
  return out.reshape(NC * NP, HF)


def _final_body(p_ref, w1_ref, b1_ref, w2_ref, b2_ref, o_ref):
  g = jnp.concatenate([p_ref[0], p_ref[1]], axis=1)
  t = jnp.dot(g, w1_ref[...], preferred_element_type=jnp.float32)
  t = jnp.maximum(t + b1_ref[...], 0.0)
  o_ref[...] = jnp.dot(t, w2_ref[...], preferred_element_type=jnp.float32) \
      + b2_ref[...]


def _final(parts, W1, b1, W2, b2):
  return pl.pallas_call(
      _final_body,
      out_shape=jax.ShapeDtypeStruct((GG, 1), jnp.float32),
  )(parts, W1, b1.reshape(1, DD), W2, b2.reshape(1, 1))


@jax.jit
def kernel(x, edge_index, batch, params):
  src2 = edge_index[0].reshape(EE // ECH, ECH)
  dst2 = edge_index[1].reshape(EE // ECH, ECH)
  zpad = jnp.zeros((NP - NN, HF), jnp.float32)
  h2 = jnp.concatenate([x[:, :HF], zpad, x[:, HF:], zpad], axis=0)
  for (W1, b1, W2, b2) in params["convs"]:
    z2 = _agg(h2, src2, dst2)
    h2 = _mlp(z2, W1, b1, W2, b2)
  parts = _pool(h2, batch)
  out = _final(parts, params["fc_W1"], params["fc_b1"],
               params["fc_W2"], params["fc_b2"])
  return out[:, 0]


# speedup vs baseline: 1.4477x; 1.0061x over previous
import functools

import jax
import jax.numpy as jnp
from jax import lax
from jax.experimental import pallas as pl
from jax.experimental.pallas import tpu as pltpu
from jax.experimental.pallas import tpu_sc as plsc

NN = 50000
EE = 800000
DD = 64
HF = DD // 2
GG = 128
NC = 2
NS = 16
LANES = 16

NP = 51200
ECH = 80
SB = 5
CPT = EE // ECH // NS
SBT = CPT // SB
RPT = NN // NS
IWCH = RPT // 5

_MESH = plsc.VectorSubcoreMesh(
    core_axis_name="c", subcore_axis_name="s", num_cores=NC, num_subcores=NS)
_SC_PARAMS = pltpu.CompilerParams(use_tc_tiling_on_sc=False)


def _agg_body(h_hbm, src_hbm, dst_hbm, out_hbm, acc, idxs, idxd, rows,
              gsem, ssem, isem):
  c = lax.axis_index("c")
  s = lax.axis_index("s")
  cbase = c * NP

  for k in range(RPT // IWCH):
    pltpu.sync_copy(
        h_hbm.at[pl.ds(cbase + s * RPT + k * IWCH, IWCH)],
        acc.at[pl.ds(s * RPT + k * IWCH, IWCH)])
  plsc.subcore_barrier()

  cb = s * CPT

  def load_idx_async(t, slot):
    pltpu.async_copy(src_hbm.at[pl.ds(cb + t * SB, SB)], idxs.at[slot], isem)
    pltpu.async_copy(dst_hbm.at[pl.ds(cb + t * SB, SB)], idxd.at[slot], isem)

  def wait_idx(slot):
    pltpu.make_async_copy(src_hbm.at[pl.ds(0, SB)], idxs.at[slot],
                          isem).wait()
    pltpu.make_async_copy(src_hbm.at[pl.ds(0, SB)], idxd.at[slot],
                          isem).wait()

  def shift_idx(slot):
    for j in range(SB):
      for k in range(ECH // LANES):
        v = idxs[slot, j, pl.ds(k * LANES, LANES)]
        idxs[slot, j, pl.ds(k * LANES, LANES)] = v + cbase

  def fire_gathers(islot, rslot):
    for j in range(SB):
      pltpu.async_copy(h_hbm.at[pl.ds(cbase, NP)].at[idxs.at[islot, j]],
                      rows.at[rslot, j], gsem)

  def drain_gather(rslot, j):
    pltpu.make_async_copy(h_hbm.at[pl.ds(0, ECH)], rows.at[rslot, j],
                          gsem).wait()

  def drain_scatter(rslot, j):
    pltpu.make_async_copy(rows.at[rslot, j], acc.at[pl.ds(0, ECH)],
                          ssem).wait()

  load_idx_async(0, 0)
  wait_idx(0)
  fire_gathers(0, 0)
  load_idx_async(1, 1)

  def sb_body(t, carry):
    p = lax.rem(t, 2)
    i0 = lax.rem(t, 3)
    i1 = lax.rem(t + 1, 3)
    i2 = lax.rem(t + 2, 3)

    @pl.when(t >= 1)
    def _():
      for j in range(SB):
        drain_scatter(1 - p, j)

    @pl.when(t < SBT - 1)
    def _():
      wait_idx(i1)
      fire_gathers(i1, 1 - p)

    @pl.when(t < SBT - 2)
    def _():
      load_idx_async(t + 2, i2)

    for j in range(SB):
      drain_gather(p, j)
      pltpu.async_copy(rows.at[p, j], acc.at[idxd.at[i0, j]], ssem, add=True)
    return carry

  lax.fori_loop(0, SBT, sb_body, 0)
  for j in range(SB):
    drain_scatter((SBT - 1) % 2, j)
  plsc.subcore_barrier()

  for k in range(RPT // IWCH):
    pltpu.sync_copy(
        acc.at[pl.ds(s * RPT + k * IWCH, IWCH)],
        out_hbm.at[pl.ds(cbase + s * RPT + k * IWCH, IWCH)])


_agg = pl.kernel(
    _agg_body,
    out_type=jax.ShapeDtypeStruct((NC * NP, HF), jnp.float32),
    mesh=_MESH,
    compiler_params=_SC_PARAMS,
    scratch_types=[
        pltpu.VMEM_SHARED((NN, HF), jnp.float32),
        pltpu.VMEM((3, SB, ECH), jnp.int32),
        pltpu.VMEM((3, SB, ECH), jnp.int32),
        pltpu.VMEM((2, SB, ECH, HF), jnp.float32),
        pltpu.SemaphoreType.DMA,
        pltpu.SemaphoreType.DMA,
        pltpu.SemaphoreType.DMA,
    ],
)

PCH = 80
NPCH = NN // PCH
GPT = GG // NS


def _pool_body(h_hbm, batch_hbm, out_hbm, acc, idx_b, rows):
  c = lax.axis_index("c")
  s = lax.axis_index("s")

  zero = jnp.zeros((LANES,), jnp.float32)
  for i in range(GPT):
    for k in range(HF // LANES):
      rows[i, pl.ds(k * LANES, LANES)] = zero
  pltpu.sync_copy(rows.at[pl.ds(0, GPT)], acc.at[pl.ds(s * GPT, GPT)])
  plsc.subcore_barrier()

  def rchunk(i, carry):
    q = s * 40 + i

    @pl.when(q < NPCH)
    def _():
      pltpu.sync_copy(h_hbm.at[pl.ds(c * NP + q * PCH, PCH)], rows)
      pltpu.sync_copy(batch_hbm.at[pl.ds(q * PCH, PCH)], idx_b)
      pltpu.sync_copy(rows, acc.at[idx_b], add=True)

    return carry

  lax.fori_loop(0, 40, rchunk, 0)
  plsc.subcore_barrier()

  @pl.when(s == 0)
  def _():
    pltpu.sync_copy(acc, out_hbm.at[c])


_pool = pl.kernel(
    _pool_body,
    out_type=jax.ShapeDtypeStruct((NC, GG, HF), jnp.float32),
    mesh=_MESH,
    compiler_params=_SC_PARAMS,
    scratch_types=[
        pltpu.VMEM_SHARED((GG, HF), jnp.float32),
        pltpu.VMEM((PCH,), jnp.int32),
        pltpu.VMEM((PCH, HF), jnp.float32),
    ],
)

NQ = NC * NP * HF // 128
QB = 512
NRB = NQ // NC // QB


def _mlp_body(zl_ref, zh_ref, w1_ref, b1_ref, w2l_ref, w2h_ref,
              b2l_ref, b2h_ref, o_ref):
  j = pl.program_id(1)
  z = jnp.concatenate([zl_ref[...], zh_ref[...]], axis=1)
  t = jnp.dot(z, w1_ref[...], preferred_element_type=jnp.float32)
  t = jnp.maximum(t + b1_ref[...], 0.0)
  w2 = jnp.where(j == 0, w2l_ref[...], w2h_ref[...])
  b2 = jnp.where(j == 0, b2l_ref[...], b2h_ref[...])
  o = jnp.dot(t, w2, preferred_element_type=jnp.float32)
  o_ref[...] = jnp.maximum(o + b2, 0.0)


def _bd4(m):
  return jnp.kron(jnp.eye(4, dtype=m.dtype), m)


def _tile4(v):
  return jnp.tile(v, 4).reshape(1, 128)


def _mlp(z2, W1, b1, W2, b2):
  zv = z2.reshape(NQ, 128)
  full = lambda i, j: (0, 0)
  w1big = jnp.concatenate([
      jnp.concatenate([_bd4(W1[:HF, :HF]), _bd4(W1[:HF, HF:])], axis=1),
      jnp.concatenate([_bd4(W1[HF:, :HF]), _bd4(W1[HF:, HF:])], axis=1),
  ], axis=0)
  b1big = jnp.concatenate([_tile4(b1[:HF]), _tile4(b1[HF:])], axis=1)
  w2lo = jnp.concatenate([_bd4(W2[:HF, :HF]), _bd4(W2[HF:, :HF])], axis=0)
  w2hi = jnp.concatenate([_bd4(W2[:HF, HF:]), _bd4(W2[HF:, HF:])], axis=0)
  out = pl.pallas_call(
      _mlp_body,
      grid=(NRB, NC),
      in_specs=[
          pl.BlockSpec((QB, 128), lambda i, j: (i, 0)),
          pl.BlockSpec((QB, 128), lambda i, j: (NRB + i, 0)),
          pl.BlockSpec((256, 256), full),
          pl.BlockSpec((1, 256), full),
          pl.BlockSpec((256, 128), full),
          pl.BlockSpec((256, 128), full),
          pl.BlockSpec((1, 128), full),
          pl.BlockSpec((1, 128), full),
      ],
      out_specs=pl.BlockSpec((QB, 128), lambda i, j: (j * NRB + i, 0)),
      out_shape=jax.ShapeDtypeStruct((NQ, 128), jnp.float32),
  )(zv, zv, w1big, b1big, w2lo, w2hi, _tile4(b2[:HF]), _tile4(b2[HF:]))
  return out.reshape(NC * NP, HF)


def _final_body(p_ref, w1_ref, b1_ref, w2_ref, b2_ref, o_ref):
  g = jnp.concatenate([p_ref[0], p_ref[1]], axis=1)
  t = jnp.dot(g, w1_ref[...], preferred_element_type=jnp.float32)
  t = jnp.maximum(t + b1_ref[...], 0.0)
  o_ref[...] = jnp.dot(t, w2_ref[...], preferred_element_type=jnp.float32) \
      + b2_ref[...]


def _final(parts, W1, b1, W2, b2):
  return pl.pallas_call(
      _final_body,
      out_shape=jax.ShapeDtypeStruct((GG, 1), jnp.float32),
  )(parts, W1, b1.reshape(1, DD), W2, b2.reshape(1, 1))


@jax.jit
def kernel(x, edge_index, batch, params):
  src2 = edge_index[0].reshape(EE // ECH, ECH)
  dst2 = edge_index[1].reshape(EE // ECH, ECH)
  zpad = jnp.zeros((NP - NN, HF), jnp.float32)
  h2 = jnp.concatenate([x[:, :HF], zpad, x[:, HF:], zpad], axis=0)
  for (W1, b1, W2, b2) in params["convs"]:
    z2 = _agg(h2, src2, dst2)
    h2 = _mlp(z2, W1, b1, W2, b2)
  parts = _pool(h2, batch)
  out = _final(parts, params["fc_W1"], params["fc_b1"],
               params["fc_W2"], params["fc_b2"])
  return out[:, 0]


# speedup vs baseline: 1.5488x; 1.0699x over previous
import functools

import jax
import jax.numpy as jnp
from jax import lax
from jax.experimental import pallas as pl
from jax.experimental.pallas import tpu as pltpu
from jax.experimental.pallas import tpu_sc as plsc

NN = 50000
EE = 800000
DD = 64
HF = DD // 2
GG = 128
NC = 2
NS = 16
LANES = 16

NP = 51200
ECH = 80
SB = 5
CPT = EE // ECH // NS
SBT = CPT // SB
RPT = NN // NS
IWCH = RPT // 5

_MESH = plsc.VectorSubcoreMesh(
    core_axis_name="c", subcore_axis_name="s", num_cores=NC, num_subcores=NS)
_SC_PARAMS = pltpu.CompilerParams(use_tc_tiling_on_sc=False)


def _agg_body(h_hbm, src_hbm, dst_hbm, out_hbm, acc, idxs, idxd, rows,
              gsem, ssem, isem):
  c = lax.axis_index("c")
  s = lax.axis_index("s")
  cbase = c * NP

  for k in range(RPT // IWCH):
    pltpu.sync_copy(
        h_hbm.at[pl.ds(cbase + s * RPT + k * IWCH, IWCH)],
        acc.at[pl.ds(s * RPT + k * IWCH, IWCH)])
  plsc.subcore_barrier()

  cb = s * CPT

  def load_idx_async(t, slot):
    pltpu.async_copy(src_hbm.at[pl.ds(cb + t * SB, SB)], idxs.at[slot], isem)
    pltpu.async_copy(dst_hbm.at[pl.ds(cb + t * SB, SB)], idxd.at[slot], isem)

  def wait_idx(slot):
    pltpu.make_async_copy(src_hbm.at[pl.ds(0, SB)], idxs.at[slot],
                          isem).wait()
    pltpu.make_async_copy(src_hbm.at[pl.ds(0, SB)], idxd.at[slot],
                          isem).wait()

  def shift_idx(slot):
    for j in range(SB):
      for k in range(ECH // LANES):
        v = idxs[slot, j, pl.ds(k * LANES, LANES)]
        idxs[slot, j, pl.ds(k * LANES, LANES)] = v + cbase

  def fire_gathers(islot, rslot):
    for j in range(SB):
      pltpu.async_copy(h_hbm.at[pl.ds(cbase, NP)].at[idxs.at[islot, j]],
                      rows.at[rslot, j], gsem)

  def drain_gather(rslot, j):
    pltpu.make_async_copy(h_hbm.at[pl.ds(0, ECH)], rows.at[rslot, j],
                          gsem).wait()

  def drain_scatter(rslot, j):
    pltpu.make_async_copy(rows.at[rslot, j], acc.at[pl.ds(0, ECH)],
                          ssem).wait()

  load_idx_async(0, 0)
  wait_idx(0)
  fire_gathers(0, 0)
  load_idx_async(1, 1)

  def sb_body(t, carry):
    p = lax.rem(t, 2)
    i0 = lax.rem(t, 3)
    i1 = lax.rem(t + 1, 3)
    i2 = lax.rem(t + 2, 3)

    @pl.when(t >= 1)
    def _():
      for j in range(SB):
        drain_scatter(1 - p, j)

    @pl.when(t < SBT - 1)
    def _():
      wait_idx(i1)
      fire_gathers(i1, 1 - p)

    @pl.when(t < SBT - 2)
    def _():
      load_idx_async(t + 2, i2)

    for j in range(SB):
      drain_gather(p, j)
      pltpu.async_copy(rows.at[p, j], acc.at[idxd.at[i0, j]], ssem, add=True)
    return carry

  lax.fori_loop(0, SBT, sb_body, 0)
  for j in range(SB):
    drain_scatter((SBT - 1) % 2, j)
  plsc.subcore_barrier()

  for k in range(RPT // IWCH):
    pltpu.sync_copy(
        acc.at[pl.ds(s * RPT + k * IWCH, IWCH)],
        out_hbm.at[pl.ds(cbase + s * RPT + k * IWCH, IWCH)])


_agg = pl.kernel(
    _agg_body,
    out_type=jax.ShapeDtypeStruct((NC * NP, HF), jnp.float32),
    mesh=_MESH,
    compiler_params=_SC_PARAMS,
    scratch_types=[
        pltpu.VMEM_SHARED((NN, HF), jnp.float32),
        pltpu.VMEM((3, SB, ECH), jnp.int32),
        pltpu.VMEM((3, SB, ECH), jnp.int32),
        pltpu.VMEM((2, SB, ECH, HF), jnp.float32),
        pltpu.SemaphoreType.DMA,
        pltpu.SemaphoreType.DMA,
        pltpu.SemaphoreType.DMA,
    ],
)

NQ = NC * NP * HF // 128
QB = 512
NRB = NQ // NC // QB


def _mlp_body(zl_ref, zh_ref, w1_ref, b1_ref, w2l_ref, w2h_ref,
              b2l_ref, b2h_ref, o_ref):
  j = pl.program_id(1)
  z = jnp.concatenate([zl_ref[...], zh_ref[...]], axis=1)
  t = jnp.dot(z, w1_ref[...], preferred_element_type=jnp.float32)
  t = jnp.maximum(t + b1_ref[...], 0.0)
  w2 = jnp.where(j == 0, w2l_ref[...], w2h_ref[...])
  b2 = jnp.where(j == 0, b2l_ref[...], b2h_ref[...])
  o = jnp.dot(t, w2, preferred_element_type=jnp.float32)
  o_ref[...] = jnp.maximum(o + b2, 0.0)


HQ = NP * HF // 128
HQREAL = NN * HF // 128


def _mlp_pool_body(zl_ref, zh_ref, w1_ref, b1_ref, w2l_ref, w2h_ref,
                   b2l_ref, b2h_ref, bcol_ref, g_ref):
  i = pl.program_id(0)
  j = pl.program_id(1)
  z = jnp.concatenate([zl_ref[...], zh_ref[...]], axis=1)
  t = jnp.dot(z, w1_ref[...], preferred_element_type=jnp.float32)
  t = jnp.maximum(t + b1_ref[...], 0.0)
  w2 = jnp.where(j == 0, w2l_ref[...], w2h_ref[...])
  b2 = jnp.where(j == 0, b2l_ref[...], b2h_ref[...])
  o = jnp.dot(t, w2, preferred_element_type=jnp.float32)
  o = jnp.maximum(o + b2, 0.0)

  rowid = lax.broadcasted_iota(jnp.int32, (QB, 128), 0)
  om = jnp.where(rowid < HQREAL - i * QB, o, 0.0)
  giota = lax.broadcasted_iota(jnp.int32, (QB, GG), 1)
  colid = lax.broadcasted_iota(jnp.int32, (GG, 128), 1)
  bcol = bcol_ref[...]
  gacc = jnp.zeros((GG, 128), jnp.float32)
  for a in range(4):
    sa = (bcol[:, a:a + 1] == giota).astype(jnp.float32)
    ga = lax.dot_general(sa, om, (((0,), (0,)), ((), ())),
                         preferred_element_type=jnp.float32)
    gacc = gacc + jnp.where((colid // HF) == a, ga, 0.0)

  @pl.when(i == 0)
  def _():
    g_ref[0] = gacc

  @pl.when(i > 0)
  def _():
    g_ref[0] = g_ref[0] + gacc


def _bd4(m):
  return jnp.kron(jnp.eye(4, dtype=m.dtype), m)


def _tile4(v):
  return jnp.tile(v, 4).reshape(1, 128)


def _packed_weights(W1, b1, W2, b2):
  w1big = jnp.concatenate([
      jnp.concatenate([_bd4(W1[:HF, :HF]), _bd4(W1[:HF, HF:])], axis=1),
      jnp.concatenate([_bd4(W1[HF:, :HF]), _bd4(W1[HF:, HF:])], axis=1),
  ], axis=0)
  b1big = jnp.concatenate([_tile4(b1[:HF]), _tile4(b1[HF:])], axis=1)
  w2lo = jnp.concatenate([_bd4(W2[:HF, :HF]), _bd4(W2[HF:, :HF])], axis=0)
  w2hi = jnp.concatenate([_bd4(W2[:HF, HF:]), _bd4(W2[HF:, HF:])], axis=0)
  return w1big, b1big, w2lo, w2hi, _tile4(b2[:HF]), _tile4(b2[HF:])


_FULL = lambda i, j: (0, 0)
_W_SPECS = [
    pl.BlockSpec((256, 256), _FULL),
    pl.BlockSpec((1, 256), _FULL),
    pl.BlockSpec((256, 128), _FULL),
    pl.BlockSpec((256, 128), _FULL),
    pl.BlockSpec((1, 128), _FULL),
    pl.BlockSpec((1, 128), _FULL),
]
_Z_SPECS = [
    pl.BlockSpec((QB, 128), lambda i, j: (i, 0)),
    pl.BlockSpec((QB, 128), lambda i, j: (NRB + i, 0)),
]


def _mlp(z2, W1, b1, W2, b2):
  zv = z2.reshape(NQ, 128)
  out = pl.pallas_call(
      _mlp_body,
      grid=(NRB, NC),
      in_specs=_Z_SPECS + _W_SPECS,
      out_specs=pl.BlockSpec((QB, 128), lambda i, j: (j * NRB + i, 0)),
      out_shape=jax.ShapeDtypeStruct((NQ, 128), jnp.float32),
  )(zv, zv, *_packed_weights(W1, b1, W2, b2))
  return out.reshape(NC * NP, HF)


def _mlp_pool(z2, W1, b1, W2, b2, bcol):
  zv = z2.reshape(NQ, 128)
  return pl.pallas_call(
      _mlp_pool_body,
      grid=(NRB, NC),
      in_specs=_Z_SPECS + _W_SPECS + [
          pl.BlockSpec((QB, 4), lambda i, j: (i, 0))],
      out_specs=pl.BlockSpec((1, GG, 128), lambda i, j: (j, 0, 0)),
      out_shape=jax.ShapeDtypeStruct((NC, GG, 128), jnp.float32),
  )(zv, zv, *_packed_weights(W1, b1, W2, b2), bcol)


def _final_body(p_ref, w1_ref, b1_ref, w2_ref, b2_ref, o_ref):
  gl = (p_ref[0][:, :HF] + p_ref[0][:, HF:2 * HF]
        + p_ref[0][:, 2 * HF:3 * HF] + p_ref[0][:, 3 * HF:])
  gh = (p_ref[1][:, :HF] + p_ref[1][:, HF:2 * HF]
        + p_ref[1][:, 2 * HF:3 * HF] + p_ref[1][:, 3 * HF:])
  g = jnp.concatenate([gl, gh], axis=1)
  t = jnp.dot(g, w1_ref[...], preferred_element_type=jnp.float32)
  t = jnp.maximum(t + b1_ref[...], 0.0)
  o_ref[...] = jnp.dot(t, w2_ref[...], preferred_element_type=jnp.float32) \
      + b2_ref[...]


def _final(parts, W1, b1, W2, b2):
  return pl.pallas_call(
      _final_body,
      out_shape=jax.ShapeDtypeStruct((GG, 1), jnp.float32),
  )(parts, W1, b1.reshape(1, DD), W2, b2.reshape(1, 1))


@jax.jit
def kernel(x, edge_index, batch, params):
  src2 = edge_index[0].reshape(EE // ECH, ECH)
  dst2 = edge_index[1].reshape(EE // ECH, ECH)
  zpad = jnp.zeros((NP - NN, HF), jnp.float32)
  h2 = jnp.concatenate([x[:, :HF], zpad, x[:, HF:], zpad], axis=0)
  bcol = jnp.concatenate(
      [batch, jnp.full((NP - NN,), GG, jnp.int32)]).reshape(HQ, 4)
  convs = params["convs"]
  for (W1, b1, W2, b2) in convs[:-1]:
    z2 = _agg(h2, src2, dst2)
    h2 = _mlp(z2, W1, b1, W2, b2)
  (W1, b1, W2, b2) = convs[-1]
  z2 = _agg(h2, src2, dst2)
  parts = _mlp_pool(z2, W1, b1, W2, b2, bcol)
  out = _final(parts, params["fc_W1"], params["fc_b1"],
               params["fc_W2"], params["fc_b2"])
  return out[:, 0]
